# Initial kernel scaffold; baseline (speedup 1.0000x reference)
import jax
import jax.numpy as jnp
from jax.experimental import pallas as pl


def kernel(input_ids, position_ids, wte, wpe):
    raise NotImplementedError("write your pallas kernel here")


# speedup vs baseline: 1.5772x; 1.5772x over previous
import functools

import jax
import jax.numpy as jnp
from jax import lax
from jax.experimental import pallas as pl
from jax.experimental.pallas import tpu as pltpu
from jax.experimental.pallas import tpu_sc as plsc

D = 128
N_TOK = 8192
NC = 2
NS = 16
NW = NC * NS
PER_W = N_TOK // NW
CHUNK = 128
LANES = 16

_mesh = plsc.VectorSubcoreMesh(core_axis_name="c", subcore_axis_name="s")


@functools.partial(
    pl.kernel,
    out_type=jax.ShapeDtypeStruct((N_TOK, D), jnp.float32),
    mesh=_mesh,
    scratch_types=[
        pltpu.VMEM((CHUNK,), jnp.int32),
        pltpu.VMEM((CHUNK,), jnp.int32),
        pltpu.VMEM((CHUNK, D), jnp.float32),
        pltpu.VMEM((CHUNK, D), jnp.float32),
        pltpu.SemaphoreType.DMA,
        pltpu.SemaphoreType.DMA,
    ],
)
def _embed(ids_hbm, pos_hbm, wte_hbm, wpe_hbm, out_hbm,
           tid_v, pid_v, a_v, b_v, s1, s2):
    wid = lax.axis_index("s") * NC + lax.axis_index("c")
    for chunk in range(PER_W // CHUNK):
        base = wid * PER_W + chunk * CHUNK
        pltpu.sync_copy(ids_hbm.at[pl.ds(base, CHUNK)], tid_v)
        pltpu.sync_copy(pos_hbm.at[pl.ds(base, CHUNK)], pid_v)
        ca = pltpu.async_copy(wte_hbm.at[tid_v], a_v, s1)
        cb = pltpu.async_copy(wpe_hbm.at[pid_v], b_v, s2)
        ca.wait()
        cb.wait()

        def row(i, carry):
            for j in range(D // LANES):
                sl = pl.ds(j * LANES, LANES)
                plsc.addupdate(a_v.at[i, sl], b_v[i, sl])
            return carry

        lax.fori_loop(0, CHUNK, row, 0)
        pltpu.sync_copy(a_v, out_hbm.at[pl.ds(base, CHUNK)])


def kernel(input_ids, position_ids, wte, wpe):
    ids = input_ids.reshape(-1).astype(jnp.int32)
    pos = position_ids.reshape(-1).astype(jnp.int32)
    out = _embed(ids, pos, wte, wpe)
    return out.reshape(input_ids.shape + (wte.shape[1],))


# speedup vs baseline: 1.7245x; 1.0934x over previous
import functools

import jax
import jax.numpy as jnp
from jax import lax
from jax.experimental import pallas as pl
from jax.experimental.pallas import tpu as pltpu
from jax.experimental.pallas import tpu_sc as plsc

D = 128
N_TOK = 8192
NC = 2
NS = 16
NW = NC * NS
PER_W = N_TOK // NW
CHUNK = 128
LANES = 16

_mesh = plsc.VectorSubcoreMesh(core_axis_name="c", subcore_axis_name="s")


def _add_rows(a, b):
    def body(i, carry):
        for r in range(4):
            row = i * 4 + r
            for j in range(D // LANES):
                sl = pl.ds(j * LANES, LANES)
                plsc.addupdate(a.at[row, sl], b[row, sl])
        return carry

    lax.fori_loop(0, CHUNK // 4, body, 0)


@functools.partial(
    pl.kernel,
    out_type=jax.ShapeDtypeStruct((N_TOK, D), jnp.float32),
    mesh=_mesh,
    scratch_types=[
        pltpu.VMEM((2, CHUNK), jnp.int32),
        pltpu.VMEM((2, CHUNK), jnp.int32),
        pltpu.VMEM((CHUNK, D), jnp.float32),
        pltpu.VMEM((CHUNK, D), jnp.float32),
        pltpu.VMEM((CHUNK, D), jnp.float32),
        pltpu.VMEM((CHUNK, D), jnp.float32),
        pltpu.SemaphoreType.DMA,
        pltpu.SemaphoreType.DMA,
        pltpu.SemaphoreType.DMA,
        pltpu.SemaphoreType.DMA,
        pltpu.SemaphoreType.DMA,
        pltpu.SemaphoreType.DMA,
        pltpu.SemaphoreType.DMA,
    ],
)
def _embed(idx_hbm, wte_hbm, wpe_hbm, out_hbm,
           ti_v, pi_v, a0, b0, a1, b1,
           si0, si1, sa0, sb0, sa1, sb1, so):
    wid = lax.axis_index("s") * NC + lax.axis_index("c")
    r = wid * 2
    ci0 = pltpu.async_copy(idx_hbm.at[pl.ds(r, 2)], ti_v, si0)
    ci1 = pltpu.async_copy(idx_hbm.at[pl.ds(NW * 2 + r, 2)], pi_v, si1)
    ci0.wait()
    ci1.wait()
    ga0 = pltpu.async_copy(wte_hbm.at[ti_v.at[0]], a0, sa0)
    gb0 = pltpu.async_copy(wpe_hbm.at[pi_v.at[0]], b0, sb0)
    ga1 = pltpu.async_copy(wte_hbm.at[ti_v.at[1]], a1, sa1)
    gb1 = pltpu.async_copy(wpe_hbm.at[pi_v.at[1]], b1, sb1)
    base = wid * PER_W
    ga0.wait()
    gb0.wait()
    _add_rows(a0, b0)
    co0 = pltpu.async_copy(a0, out_hbm.at[pl.ds(base, CHUNK)], so)
    ga1.wait()
    gb1.wait()
    _add_rows(a1, b1)
    co0.wait()
    co1 = pltpu.async_copy(a1, out_hbm.at[pl.ds(base + CHUNK, CHUNK)], so)
    co1.wait()


def kernel(input_ids, position_ids, wte, wpe):
    idx = jnp.concatenate(
        [input_ids.reshape(-1), position_ids.reshape(-1)]
    ).astype(jnp.int32).reshape(2 * NW * 2, CHUNK)
    out = _embed(idx, wte, wpe)
    return out.reshape(input_ids.shape + (wte.shape[1],))
